# baseline (speedup 1.0000x reference)
import functools

import jax
import jax.numpy as jnp
from jax.experimental import pallas as pl
from jax.experimental.pallas import tpu as pltpu

N_NODES = 100000
N_EDGES = 1600000
D_FEAT = 128
D_EDGE = 16
D_CTX = 128
D_OUT = 128

GRID = 250
BV = N_NODES // GRID
BE = N_EDGES // GRID


def _body(ctx_ref, v_ref, e_ref, w_ref, b_ref, out_ref, vacc, eacc):
    i = pl.program_id(0)

    @pl.when(i == 0)
    def _init():
        vacc[...] = jnp.zeros_like(vacc)
        eacc[...] = jnp.zeros_like(eacc)

    vacc[...] += jnp.sum(v_ref[...].reshape(BV // 8, 8, D_FEAT), axis=0)
    eacc[...] += jnp.sum(e_ref[...].reshape(BE // 8, 8, D_EDGE), axis=0)

    @pl.when(i == GRID - 1)
    def _fini():
        v_mean = jnp.sum(vacc[...], axis=0, keepdims=True) / N_NODES
        e_mean = jnp.sum(eacc[...], axis=0, keepdims=True) / N_EDGES
        x = jnp.concatenate([ctx_ref[...], v_mean, e_mean], axis=1)
        out_ref[...] = jnp.dot(x, w_ref[...],
                               preferred_element_type=jnp.float32) + b_ref[...]


def kernel(context, vertex_data, edge_data, W, b):
    b2 = b.reshape(1, D_OUT)
    out = pl.pallas_call(
        _body,
        grid=(GRID,),
        in_specs=[
            pl.BlockSpec((1, D_CTX), lambda i: (0, 0)),
            pl.BlockSpec((BV, D_FEAT), lambda i: (i, 0)),
            pl.BlockSpec((BE, D_EDGE), lambda i: (i, 0)),
            pl.BlockSpec((D_CTX + D_FEAT + D_EDGE, D_OUT), lambda i: (0, 0)),
            pl.BlockSpec((1, D_OUT), lambda i: (0, 0)),
        ],
        out_specs=pl.BlockSpec((1, D_OUT), lambda i: (0, 0)),
        out_shape=jax.ShapeDtypeStruct((1, D_OUT), jnp.float32),
        scratch_shapes=[
            pltpu.VMEM((8, D_FEAT), jnp.float32),
            pltpu.VMEM((8, D_EDGE), jnp.float32),
        ],
    )(context, vertex_data, edge_data, W, b2)
    return out
